# baseline (speedup 1.0000x reference)
import functools
import math

import jax
import jax.numpy as jnp
from jax import lax
from jax.experimental import pallas as pl
from jax.experimental.pallas import tpu as pltpu
from jax.experimental.pallas import tpu_sc as plsc

_VOCAB = 1000
_D = 64
_DP = 128
_BATCH = 16384
_FIELDS = 26
_SCALE = math.sqrt(_D)

_NC = 2
_NS = 16
_NW = _NC * _NS
_B_PER_W = _BATCH // _NW
_NB = 8
_NCHUNK = _B_PER_W // _NB


def _scale_pad_body(t_ref, o_ref):
    o_ref[...] = jnp.zeros((_VOCAB, _DP), jnp.float32)
    o_ref[:, :_D] = t_ref[...] * _SCALE


def _scale_pad_table(table):
    return pl.pallas_call(
        _scale_pad_body,
        out_shape=jax.ShapeDtypeStruct((_VOCAB, _DP), jnp.float32),
    )(table)


_mesh = plsc.VectorSubcoreMesh(core_axis_name="c", subcore_axis_name="s")


@functools.partial(
    pl.kernel,
    mesh=_mesh,
    out_type=jax.ShapeDtypeStruct((_BATCH, _FIELDS, _D), jnp.float32),
    scratch_types=[
        pltpu.VMEM((_NB, _FIELDS), jnp.int32),
        pltpu.VMEM((_NB, _FIELDS, _DP), jnp.float32),
        pltpu.VMEM((_NB, _FIELDS, _D), jnp.float32),
        pltpu.SemaphoreType.DMA,
    ],
)
def _gather_kernel(x_hbm, tab_hbm, out_hbm, idx_v, raw_v, planes_v, sem):
    wid = lax.axis_index("s") * _NC + lax.axis_index("c")
    base = wid * _B_PER_W

    def chunk(g, carry):
        b0 = base + g * _NB
        pltpu.sync_copy(x_hbm.at[pl.ds(b0, _NB)], idx_v)
        copies = []
        for k in range(_NB):
            copies.append(
                pltpu.async_copy(tab_hbm.at[idx_v.at[k]], raw_v.at[k], sem)
            )
        for c in copies:
            c.wait()

        def repack(n, carry2):
            for f in range(_FIELDS):
                for c4 in range(_D // 16):
                    sl = pl.ds(c4 * 16, 16)
                    planes_v[n, f, sl] = raw_v[n, f, sl]
            return carry2

        lax.fori_loop(0, _NB, repack, 0)
        pltpu.sync_copy(planes_v, out_hbm.at[pl.ds(b0, _NB)])
        return carry

    lax.fori_loop(0, _NCHUNK, chunk, 0)


def kernel(x, table):
    scaled = _scale_pad_table(table)
    return _gather_kernel(x.astype(jnp.int32), scaled)
